# baseline (speedup 1.0000x reference)
import functools

import jax
import jax.numpy as jnp
from jax import lax
from jax.experimental import pallas as pl
from jax.experimental.pallas import tpu as pltpu
from jax.experimental.pallas import tpu_sc as plsc

N_NODES = 10000
N_EDGES = 320000
D = 128

NC = 2
NS = 16
NW = NC * NS
CHUNK = 125
NCH = 80
NCHG = 40
NGRP = NCH // NCHG
N_PAD = 10240
RPT = N_PAD // NS

_mesh = plsc.VectorSubcoreMesh(core_axis_name="c", subcore_axis_name="s")


@functools.partial(
    pl.kernel,
    out_type=[
        jax.ShapeDtypeStruct((NC, N_PAD), jnp.float32),
        jax.ShapeDtypeStruct((NC, N_PAD), jnp.float32),
    ],
    mesh=_mesh,
    scratch_types=[
        pltpu.VMEM((NCH, CHUNK), jnp.int32),
        pltpu.VMEM((NCH, CHUNK), jnp.int32),
        pltpu.VMEM((128,), jnp.float32),
        pltpu.VMEM((RPT,), jnp.float32),
        pltpu.VMEM_SHARED((N_PAD,), jnp.float32),
        pltpu.VMEM_SHARED((N_PAD,), jnp.float32),
        pltpu.SemaphoreType.DMA,
        pltpu.SemaphoreType.DMA,
    ],
)
def _sc_degrees(
    s_hbm, r_hbm, sd_out, rd_out, sidx, ridx, ones, zbuf, sdeg, rdeg, dsem0, dsem1
):
    c = lax.axis_index("c")
    s = lax.axis_index("s")
    wid = c * NS + s

    pltpu.async_copy(s_hbm.at[wid], sidx, dsem0)
    pltpu.async_copy(r_hbm.at[wid], ridx, dsem1)

    def fill(i, _):
        ones[pl.ds(i * 16, 16)] = jnp.ones((16,), jnp.float32)
        return _

    lax.fori_loop(0, 8, fill, None)

    def zfill(i, _):
        zbuf[pl.ds(i * 16, 16)] = jnp.zeros((16,), jnp.float32)
        return _

    lax.fori_loop(0, RPT // 16, zfill, None)

    pltpu.make_async_copy(s_hbm.at[wid], sidx, dsem0).wait()
    pltpu.make_async_copy(r_hbm.at[wid], ridx, dsem1).wait()
    pltpu.sync_copy(zbuf, sdeg.at[pl.ds(s * RPT, RPT)])
    pltpu.sync_copy(zbuf, rdeg.at[pl.ds(s * RPT, RPT)])
    plsc.subcore_barrier()

    def body(ch, _):
        pltpu.async_copy(ones.at[pl.ds(0, CHUNK)], sdeg.at[sidx.at[ch]], dsem0, add=True)
        pltpu.async_copy(ones.at[pl.ds(0, CHUNK)], rdeg.at[ridx.at[ch]], dsem1, add=True)

        @pl.when(ch >= 1)
        def _drain():
            pltpu.make_async_copy(
                ones.at[pl.ds(0, CHUNK)], sdeg.at[sidx.at[ch - 1]], dsem0
            ).wait()
            pltpu.make_async_copy(
                ones.at[pl.ds(0, CHUNK)], rdeg.at[ridx.at[ch - 1]], dsem1
            ).wait()

        return _

    lax.fori_loop(0, NCH, body, None)
    pltpu.make_async_copy(
        ones.at[pl.ds(0, CHUNK)], sdeg.at[sidx.at[NCH - 1]], dsem0
    ).wait()
    pltpu.make_async_copy(
        ones.at[pl.ds(0, CHUNK)], rdeg.at[ridx.at[NCH - 1]], dsem1
    ).wait()
    plsc.subcore_barrier()

    pltpu.async_copy(
        sdeg.at[pl.ds(s * RPT, RPT)], sd_out.at[c, pl.ds(s * RPT, RPT)], dsem0
    )
    pltpu.async_copy(
        rdeg.at[pl.ds(s * RPT, RPT)], rd_out.at[c, pl.ds(s * RPT, RPT)], dsem1
    )
    pltpu.make_async_copy(
        sdeg.at[pl.ds(s * RPT, RPT)], sd_out.at[c, pl.ds(s * RPT, RPT)], dsem0
    ).wait()
    pltpu.make_async_copy(
        rdeg.at[pl.ds(s * RPT, RPT)], rd_out.at[c, pl.ds(s * RPT, RPT)], dsem1
    ).wait()


@functools.partial(
    pl.kernel,
    out_type=jax.ShapeDtypeStruct((NC, N_PAD, D), jnp.float32),
    mesh=_mesh,
    scratch_types=[
        pltpu.VMEM((NCHG, CHUNK), jnp.int32),
        pltpu.VMEM((NCHG, CHUNK), jnp.int32),
        pltpu.VMEM((2, CHUNK, D), jnp.float32),
        pltpu.VMEM_SHARED((N_PAD, D), jnp.float32),
        pltpu.SemaphoreType.DMA,
        pltpu.SemaphoreType.DMA,
        pltpu.SemaphoreType.DMA,
        pltpu.SemaphoreType.DMA,
    ],
)
def _sc_aggregate(g_hbm, s_hbm, r_hbm, p_out, sidx, ridx, rows, acc, g0, g1, a0, a1):
    c = lax.axis_index("c")
    s = lax.axis_index("s")
    wid = c * NS + s

    def zfill(i, _):
        for j in range(D // 16):
            rows[0, i, pl.ds(j * 16, 16)] = jnp.zeros((16,), jnp.float32)
        return _

    pltpu.async_copy(s_hbm.at[wid, pl.ds(0, NCHG)], sidx, g0)
    pltpu.async_copy(r_hbm.at[wid, pl.ds(0, NCHG)], ridx, g1)

    lax.fori_loop(0, CHUNK, zfill, None)
    nfull = RPT // CHUNK
    rem = RPT - nfull * CHUNK
    for k in range(nfull):
        pltpu.async_copy(
            rows.at[0], acc.at[pl.ds(s * RPT + k * CHUNK, CHUNK)], (a0, a1)[k % 2]
        )
    if rem:
        pltpu.async_copy(
            rows.at[0, pl.ds(0, rem)],
            acc.at[pl.ds(s * RPT + nfull * CHUNK, rem)],
            (a0, a1)[nfull % 2],
        )
    for k in range(nfull):
        pltpu.make_async_copy(
            rows.at[0], acc.at[pl.ds(s * RPT + k * CHUNK, CHUNK)], (a0, a1)[k % 2]
        ).wait()
    if rem:
        pltpu.make_async_copy(
            rows.at[0, pl.ds(0, rem)],
            acc.at[pl.ds(s * RPT + nfull * CHUNK, rem)],
            (a0, a1)[nfull % 2],
        ).wait()
    pltpu.make_async_copy(s_hbm.at[wid, pl.ds(0, NCHG)], sidx, g0).wait()
    pltpu.make_async_copy(r_hbm.at[wid, pl.ds(0, NCHG)], ridx, g1).wait()
    plsc.subcore_barrier()

    for q in range(NGRP):
        if q:
            pltpu.sync_copy(s_hbm.at[wid, pl.ds(q * NCHG, NCHG)], sidx)
            pltpu.sync_copy(r_hbm.at[wid, pl.ds(q * NCHG, NCHG)], ridx)
        pltpu.async_copy(g_hbm.at[sidx.at[0]], rows.at[0], g0)

        def body(i, _):
            for b in range(2):
                ch = i * 2 + b
                nxt = ch + 1

                @pl.when(nxt < NCHG)
                def _start():
                    pltpu.async_copy(
                        g_hbm.at[sidx.at[nxt]], rows.at[1 - b], (g0, g1)[1 - b]
                    )

                pltpu.make_async_copy(
                    g_hbm.at[sidx.at[ch]], rows.at[b], (g0, g1)[b]
                ).wait()
                pltpu.sync_copy(rows.at[b], acc.at[ridx.at[ch]], add=True)
            return _

        lax.fori_loop(0, NCHG // 2, body, None)
    plsc.subcore_barrier()

    pltpu.sync_copy(acc.at[pl.ds(s * RPT, RPT)], p_out.at[c, pl.ds(s * RPT, RPT)])


def _tc_matmul_body(x_ref, w_ref, b_ref, h_ref):
    h = jnp.dot(x_ref[...], w_ref[...], preferred_element_type=jnp.float32)
    h_ref[...] = h + b_ref[...]


def _tc_scale_body(h_ref, sd_ref, g_ref):
    sd = sd_ref[0, :] + sd_ref[1, :]
    g_ref[...] = h_ref[...] * lax.rsqrt(jnp.maximum(sd, 1.0))[:, None]


def _tc_finalize_body(p_ref, rd_ref, o_ref):
    acc = p_ref[0] + p_ref[1]
    rd = rd_ref[0, :] + rd_ref[1, :]
    o_ref[...] = acc * lax.rsqrt(jnp.maximum(rd, 1.0))[:, None]


_ROWS_BLK = 1280


def kernel(x, senders, receivers, weight, bias):
    s_pad = senders.reshape(NW, NCH, CHUNK)
    r_pad = receivers.reshape(NW, NCH, CHUNK)

    sd_p, rd_p = _sc_degrees(s_pad, r_pad)

    h = pl.pallas_call(
        _tc_matmul_body,
        grid=(N_PAD // _ROWS_BLK,),
        in_specs=[
            pl.BlockSpec((_ROWS_BLK, D), lambda i: (i, 0)),
            pl.BlockSpec((D, D), lambda i: (0, 0)),
            pl.BlockSpec((1, D), lambda i: (0, 0)),
        ],
        out_specs=pl.BlockSpec((_ROWS_BLK, D), lambda i: (i, 0)),
        out_shape=jax.ShapeDtypeStruct((N_PAD, D), jnp.float32),
    )(x, weight, bias.reshape(1, D))

    g = pl.pallas_call(
        _tc_scale_body,
        grid=(N_PAD // _ROWS_BLK,),
        in_specs=[
            pl.BlockSpec((_ROWS_BLK, D), lambda i: (i, 0)),
            pl.BlockSpec((NC, _ROWS_BLK), lambda i: (0, i)),
        ],
        out_specs=pl.BlockSpec((_ROWS_BLK, D), lambda i: (i, 0)),
        out_shape=jax.ShapeDtypeStruct((N_PAD, D), jnp.float32),
    )(h, sd_p)

    p = _sc_aggregate(g, s_pad, r_pad)

    out = pl.pallas_call(
        _tc_finalize_body,
        grid=(N_PAD // _ROWS_BLK,),
        in_specs=[
            pl.BlockSpec((NC, _ROWS_BLK, D), lambda i: (0, i, 0)),
            pl.BlockSpec((NC, _ROWS_BLK), lambda i: (0, i)),
        ],
        out_specs=pl.BlockSpec((_ROWS_BLK, D), lambda i: (i, 0)),
        out_shape=jax.ShapeDtypeStruct((N_NODES, D), jnp.float32),
    )(p, rd_p)
    return out


# speedup vs baseline: 1.0044x; 1.0044x over previous
import functools

import jax
import jax.numpy as jnp
from jax import lax
from jax.experimental import pallas as pl
from jax.experimental.pallas import tpu as pltpu
from jax.experimental.pallas import tpu_sc as plsc

N_NODES = 10000
N_EDGES = 320000
D = 128

NC = 2
NS = 16
NW = NC * NS
CHUNK = 125
NCH = 80
NCHG = 40
NGRP = NCH // NCHG
N_PAD = 10240
RPT = N_PAD // NS

_mesh = plsc.VectorSubcoreMesh(core_axis_name="c", subcore_axis_name="s")


@functools.partial(
    pl.kernel,
    out_type=[
        jax.ShapeDtypeStruct((NC, N_PAD), jnp.float32),
        jax.ShapeDtypeStruct((NC, N_PAD), jnp.float32),
    ],
    mesh=_mesh,
    scratch_types=[
        pltpu.VMEM((NCH, CHUNK), jnp.int32),
        pltpu.VMEM((NCH, CHUNK), jnp.int32),
        pltpu.VMEM((128,), jnp.float32),
        pltpu.VMEM((RPT,), jnp.float32),
        pltpu.VMEM_SHARED((N_PAD,), jnp.float32),
        pltpu.VMEM_SHARED((N_PAD,), jnp.float32),
        pltpu.SemaphoreType.DMA,
        pltpu.SemaphoreType.DMA,
    ],
)
def _sc_degrees(
    s_hbm, r_hbm, sd_out, rd_out, sidx, ridx, ones, zbuf, sdeg, rdeg, dsem0, dsem1
):
    c = lax.axis_index("c")
    s = lax.axis_index("s")
    wid = c * NS + s

    pltpu.async_copy(s_hbm.at[wid], sidx, dsem0)
    pltpu.async_copy(r_hbm.at[wid], ridx, dsem1)

    def fill(i, _):
        ones[pl.ds(i * 16, 16)] = jnp.ones((16,), jnp.float32)
        return _

    lax.fori_loop(0, 8, fill, None)

    def zfill(i, _):
        zbuf[pl.ds(i * 16, 16)] = jnp.zeros((16,), jnp.float32)
        return _

    lax.fori_loop(0, RPT // 16, zfill, None)

    pltpu.make_async_copy(s_hbm.at[wid], sidx, dsem0).wait()
    pltpu.make_async_copy(r_hbm.at[wid], ridx, dsem1).wait()
    pltpu.sync_copy(zbuf, sdeg.at[pl.ds(s * RPT, RPT)])
    pltpu.sync_copy(zbuf, rdeg.at[pl.ds(s * RPT, RPT)])
    plsc.subcore_barrier()

    def body(ch, _):
        pltpu.async_copy(ones.at[pl.ds(0, CHUNK)], sdeg.at[sidx.at[ch]], dsem0, add=True)
        pltpu.async_copy(ones.at[pl.ds(0, CHUNK)], rdeg.at[ridx.at[ch]], dsem1, add=True)

        @pl.when(ch >= 1)
        def _drain():
            pltpu.make_async_copy(
                ones.at[pl.ds(0, CHUNK)], sdeg.at[sidx.at[ch - 1]], dsem0
            ).wait()
            pltpu.make_async_copy(
                ones.at[pl.ds(0, CHUNK)], rdeg.at[ridx.at[ch - 1]], dsem1
            ).wait()

        return _

    lax.fori_loop(0, NCH, body, None)
    pltpu.make_async_copy(
        ones.at[pl.ds(0, CHUNK)], sdeg.at[sidx.at[NCH - 1]], dsem0
    ).wait()
    pltpu.make_async_copy(
        ones.at[pl.ds(0, CHUNK)], rdeg.at[ridx.at[NCH - 1]], dsem1
    ).wait()
    plsc.subcore_barrier()

    pltpu.async_copy(
        sdeg.at[pl.ds(s * RPT, RPT)], sd_out.at[c, pl.ds(s * RPT, RPT)], dsem0
    )
    pltpu.async_copy(
        rdeg.at[pl.ds(s * RPT, RPT)], rd_out.at[c, pl.ds(s * RPT, RPT)], dsem1
    )
    pltpu.make_async_copy(
        sdeg.at[pl.ds(s * RPT, RPT)], sd_out.at[c, pl.ds(s * RPT, RPT)], dsem0
    ).wait()
    pltpu.make_async_copy(
        rdeg.at[pl.ds(s * RPT, RPT)], rd_out.at[c, pl.ds(s * RPT, RPT)], dsem1
    ).wait()


@functools.partial(
    pl.kernel,
    out_type=jax.ShapeDtypeStruct((NC, N_PAD, D), jnp.float32),
    mesh=_mesh,
    scratch_types=[
        pltpu.VMEM((NCHG, CHUNK), jnp.int32),
        pltpu.VMEM((NCHG, CHUNK), jnp.int32),
        pltpu.VMEM((2, CHUNK, D), jnp.float32),
        pltpu.VMEM_SHARED((N_PAD, D), jnp.float32),
        pltpu.SemaphoreType.DMA,
        pltpu.SemaphoreType.DMA,
        pltpu.SemaphoreType.DMA,
        pltpu.SemaphoreType.DMA,
    ],
)
def _sc_aggregate(g_hbm, s_hbm, r_hbm, p_out, sidx, ridx, rows, acc, g0, g1, a0, a1):
    c = lax.axis_index("c")
    s = lax.axis_index("s")
    wid = c * NS + s

    def zfill(i, _):
        for j in range(D // 16):
            rows[0, i, pl.ds(j * 16, 16)] = jnp.zeros((16,), jnp.float32)
        return _

    pltpu.async_copy(s_hbm.at[wid, pl.ds(0, NCHG)], sidx, g0)
    pltpu.async_copy(r_hbm.at[wid, pl.ds(0, NCHG)], ridx, g1)

    lax.fori_loop(0, CHUNK, zfill, None)
    nfull = RPT // CHUNK
    rem = RPT - nfull * CHUNK
    for k in range(nfull):
        pltpu.async_copy(
            rows.at[0], acc.at[pl.ds(s * RPT + k * CHUNK, CHUNK)], (a0, a1)[k % 2]
        )
    if rem:
        pltpu.async_copy(
            rows.at[0, pl.ds(0, rem)],
            acc.at[pl.ds(s * RPT + nfull * CHUNK, rem)],
            (a0, a1)[nfull % 2],
        )
    for k in range(nfull):
        pltpu.make_async_copy(
            rows.at[0], acc.at[pl.ds(s * RPT + k * CHUNK, CHUNK)], (a0, a1)[k % 2]
        ).wait()
    if rem:
        pltpu.make_async_copy(
            rows.at[0, pl.ds(0, rem)],
            acc.at[pl.ds(s * RPT + nfull * CHUNK, rem)],
            (a0, a1)[nfull % 2],
        ).wait()
    pltpu.make_async_copy(s_hbm.at[wid, pl.ds(0, NCHG)], sidx, g0).wait()
    pltpu.make_async_copy(r_hbm.at[wid, pl.ds(0, NCHG)], ridx, g1).wait()
    plsc.subcore_barrier()

    for q in range(NGRP):
        if q:
            pltpu.sync_copy(s_hbm.at[wid, pl.ds(q * NCHG, NCHG)], sidx)
            pltpu.sync_copy(r_hbm.at[wid, pl.ds(q * NCHG, NCHG)], ridx)
        pltpu.async_copy(g_hbm.at[sidx.at[0]], rows.at[0], g0)

        def body(i, _):
            for b in range(2):
                ch = i * 2 + b
                nxt = ch + 1

                @pl.when(nxt < NCHG)
                def _start():
                    pltpu.async_copy(
                        g_hbm.at[sidx.at[nxt]], rows.at[1 - b], (g0, g1)[1 - b]
                    )

                pltpu.make_async_copy(
                    g_hbm.at[sidx.at[ch]], rows.at[b], (g0, g1)[b]
                ).wait()
                pltpu.sync_copy(rows.at[b], acc.at[ridx.at[ch]], add=True)
            return _

        lax.fori_loop(0, NCHG // 2, body, None)
    plsc.subcore_barrier()

    pltpu.sync_copy(acc.at[pl.ds(s * RPT, RPT)], p_out.at[c, pl.ds(s * RPT, RPT)])


def _tc_transform_body(x_ref, w_ref, b_ref, sd_ref, g_ref):
    h = jnp.dot(x_ref[...], w_ref[...], preferred_element_type=jnp.float32)
    h = h + b_ref[...]
    sd = sd_ref[0, :] + sd_ref[1, :]
    g_ref[...] = h * lax.rsqrt(jnp.maximum(sd, 1.0))[:, None]


def _tc_finalize_body(p_ref, rd_ref, o_ref):
    acc = p_ref[0] + p_ref[1]
    rd = rd_ref[0, :] + rd_ref[1, :]
    o_ref[...] = acc * lax.rsqrt(jnp.maximum(rd, 1.0))[:, None]


_ROWS_BLK = 1280


def kernel(x, senders, receivers, weight, bias):
    s_pad = senders.reshape(NW, NCH, CHUNK)
    r_pad = receivers.reshape(NW, NCH, CHUNK)

    sd_p, rd_p = _sc_degrees(s_pad, r_pad)

    g = pl.pallas_call(
        _tc_transform_body,
        grid=(N_PAD // _ROWS_BLK,),
        in_specs=[
            pl.BlockSpec((_ROWS_BLK, D), lambda i: (i, 0)),
            pl.BlockSpec((D, D), lambda i: (0, 0)),
            pl.BlockSpec((1, D), lambda i: (0, 0)),
            pl.BlockSpec((NC, _ROWS_BLK), lambda i: (0, i)),
        ],
        out_specs=pl.BlockSpec((_ROWS_BLK, D), lambda i: (i, 0)),
        out_shape=jax.ShapeDtypeStruct((N_PAD, D), jnp.float32),
    )(x, weight, bias.reshape(1, D), sd_p)

    p = _sc_aggregate(g, s_pad, r_pad)

    out = pl.pallas_call(
        _tc_finalize_body,
        grid=(N_PAD // _ROWS_BLK,),
        in_specs=[
            pl.BlockSpec((NC, _ROWS_BLK, D), lambda i: (0, i, 0)),
            pl.BlockSpec((NC, _ROWS_BLK), lambda i: (0, i)),
        ],
        out_specs=pl.BlockSpec((_ROWS_BLK, D), lambda i: (i, 0)),
        out_shape=jax.ShapeDtypeStruct((N_NODES, D), jnp.float32),
    )(p, rd_p)
    return out
